# baseline (speedup 1.0000x reference)
import functools

import jax
import jax.numpy as jnp
import numpy as np
from jax.experimental import pallas as pl
from jax.experimental.pallas import tpu as pltpu

_BS = 2048
_NQ = 4
_QB = _BS // _NQ


def _tri_kernel(im_ref, jm_ref, a0_ref, a1_ref, a2_ref, a3_ref, f_ref,
                w1_ref, b1_ref, w2_ref, b2_ref, out_ref, y_ref, xz_ref,
                *, nb, n, h_dim):
    t = pl.program_id(0)
    i = im_ref[t]
    j = jm_ref[t]
    sweep1 = t < nb * nb
    kb = n - (nb - 1) * _BS
    a_refs = (a0_ref, a1_ref, a2_ref, a3_ref)

    @pl.when(j == nb - 1)
    def _():
        for a in a_refs:
            a[:, kb:] = jnp.zeros((_QB, _BS - kb), jnp.float32)

    @pl.when(t == 0)
    def _():
        out_ref[...] = jnp.zeros_like(out_ref)
        y_ref[...] = jnp.zeros_like(y_ref)
        xz_ref[:, :h_dim] = jnp.dot(
            f_ref[...], w1_ref[...],
            preferred_element_type=jnp.float32).astype(jnp.bfloat16)
        xz_ref[:, h_dim:] = jnp.zeros_like(xz_ref[:, h_dim:])

    tiles = tuple(a[...].astype(jnp.bfloat16) for a in a_refs)
    rows_q = tuple(pl.ds(i * _BS + q * _QB, _QB) for q in range(_NQ))

    @pl.when(sweep1)
    def _():
        xzs = xz_ref[pl.ds(j * _BS, _BS), :]
        for q in range(_NQ):
            p = jnp.dot(tiles[q], xzs, preferred_element_type=jnp.float32)
            y_ref[rows_q[q], :] = y_ref[rows_q[q], :] + p[:, :h_dim]
            out_ref[rows_q[q], :] = out_ref[rows_q[q], :] + p[:, h_dim:]

        @pl.when(j == i)
        def _():
            rows = pl.ds(i * _BS, _BS)
            h = jnp.maximum(y_ref[rows, :] + b1_ref[...], 0.0)
            z = jnp.dot(h.astype(jnp.bfloat16), w2_ref[...],
                        preferred_element_type=jnp.float32)
            ridx = i * _BS + jax.lax.broadcasted_iota(jnp.int32, z.shape, 0)
            zb = jnp.where(ridx < n, z, 0.0).astype(jnp.bfloat16)
            xz_ref[rows, h_dim:] = zb
            accs = tuple(
                out_ref[rows_q[q], :] + jnp.dot(
                    tiles[q], zb, preferred_element_type=jnp.float32)
                for q in range(_NQ))

            @pl.when(i < nb - 1)
            def _():
                for q in range(_NQ):
                    out_ref[rows_q[q], :] = accs[q]

            @pl.when(i == nb - 1)
            def _():
                for q in range(_NQ):
                    o = accs[q] + b2_ref[...]
                    m = jnp.max(o, axis=1, keepdims=True)
                    e = o - m
                    out_ref[rows_q[q], :] = e - jnp.log(
                        jnp.sum(jnp.exp(e), axis=1, keepdims=True))

    @pl.when(jnp.logical_not(sweep1))
    def _():
        zs = xz_ref[pl.ds(j * _BS, _BS), h_dim:]
        accs = tuple(
            out_ref[rows_q[q], :] + jnp.dot(
                tiles[q], zs, preferred_element_type=jnp.float32)
            for q in range(_NQ))

        @pl.when(j < nb - 1)
        def _():
            for q in range(_NQ):
                out_ref[rows_q[q], :] = accs[q]

        @pl.when(j == nb - 1)
        def _():
            for q in range(_NQ):
                o = accs[q] + b2_ref[...]
                m = jnp.max(o, axis=1, keepdims=True)
                e = o - m
                out_ref[rows_q[q], :] = e - jnp.log(
                    jnp.sum(jnp.exp(e), axis=1, keepdims=True))


@jax.jit
def kernel(feature, adj, W1, b1, W2, b2):
    n, f_in = feature.shape
    h_dim = W1.shape[1]
    c_dim = W2.shape[1]
    nb = -(-n // _BS)
    npad = nb * _BS

    im_l, jm_l = [], []
    for i in range(nb):
        for j in range(nb):
            if j != i:
                im_l.append(i)
                jm_l.append(j)
        im_l.append(i)
        jm_l.append(i)
    for i in range(nb):
        for j in range(i + 1, nb):
            im_l.append(i)
            jm_l.append(j)
    im = jnp.asarray(np.asarray(im_l, dtype=np.int32))
    jm = jnp.asarray(np.asarray(jm_l, dtype=np.int32))
    steps = int(im.shape[0])

    feature_p = jnp.pad(feature.astype(jnp.bfloat16), ((0, npad - n), (0, 0)))
    b1_2d = b1.reshape(1, h_dim)
    b2_2d = b2.reshape(1, c_dim)

    def _mk_spec(q):
        return pl.BlockSpec(
            (_QB, _BS),
            lambda t, im_, jm_, q=q: (_NQ * im_[t] + q, jm_[t]))

    grid_spec = pltpu.PrefetchScalarGridSpec(
        num_scalar_prefetch=2,
        grid=(steps,),
        in_specs=[_mk_spec(q) for q in range(_NQ)] + [
            pl.BlockSpec((npad, f_in), lambda t, im_, jm_: (0, 0)),
            pl.BlockSpec((f_in, h_dim), lambda t, im_, jm_: (0, 0)),
            pl.BlockSpec((1, h_dim), lambda t, im_, jm_: (0, 0)),
            pl.BlockSpec((h_dim, c_dim), lambda t, im_, jm_: (0, 0)),
            pl.BlockSpec((1, c_dim), lambda t, im_, jm_: (0, 0)),
        ],
        out_specs=pl.BlockSpec((npad, c_dim), lambda t, im_, jm_: (0, 0)),
        scratch_shapes=[
            pltpu.VMEM((npad, h_dim), jnp.float32),
            pltpu.VMEM((npad, h_dim + c_dim), jnp.bfloat16),
        ],
    )

    out = pl.pallas_call(
        functools.partial(_tri_kernel, nb=nb, n=n, h_dim=h_dim),
        grid_spec=grid_spec,
        out_shape=jax.ShapeDtypeStruct((npad, c_dim), jnp.float32),
        compiler_params=pltpu.CompilerParams(
            vmem_limit_bytes=100 * 1024 * 1024),
    )(im, jm, adj, adj, adj, adj, feature_p, W1.astype(jnp.bfloat16),
      b1_2d, W2.astype(jnp.bfloat16), b2_2d)

    return out[:n]


# speedup vs baseline: 1.0111x; 1.0111x over previous
import functools

import jax
import jax.numpy as jnp
import numpy as np
from jax.experimental import pallas as pl
from jax.experimental.pallas import tpu as pltpu

_BS = 2048
_NQ = 4
_QB = _BS // _NQ


def _tri_kernel(im_ref, jm_ref, a0_ref, a1_ref, a2_ref, a3_ref, f_ref,
                w1_ref, b1_ref, w2_ref, b2_ref, out_ref, y_ref, xz_ref,
                *, nb, n, h_dim):
    t = pl.program_id(0)
    i = im_ref[t]
    j = jm_ref[t]
    sweep1 = t < nb * nb
    a_refs = (a0_ref, a1_ref, a2_ref, a3_ref)


    @pl.when(t == 0)
    def _():
        out_ref[...] = jnp.zeros_like(out_ref)
        y_ref[...] = jnp.zeros_like(y_ref)
        xz_ref[:, :h_dim] = jnp.dot(
            f_ref[...], w1_ref[...],
            preferred_element_type=jnp.float32).astype(jnp.bfloat16)
        xz_ref[:, h_dim:] = jnp.zeros_like(xz_ref[:, h_dim:])

    tiles = tuple(a[...].astype(jnp.bfloat16) for a in a_refs)
    rows_q = tuple(pl.ds(i * _BS + q * _QB, _QB) for q in range(_NQ))

    @pl.when(sweep1)
    def _():
        xzs = xz_ref[pl.ds(j * _BS, _BS), :]
        for q in range(_NQ):
            p = jnp.dot(tiles[q], xzs, preferred_element_type=jnp.float32)
            y_ref[rows_q[q], :] = y_ref[rows_q[q], :] + p[:, :h_dim]
            out_ref[rows_q[q], :] = out_ref[rows_q[q], :] + p[:, h_dim:]

        @pl.when(j == i)
        def _():
            rows = pl.ds(i * _BS, _BS)
            h = jnp.maximum(y_ref[rows, :] + b1_ref[...], 0.0)
            z = jnp.dot(h.astype(jnp.bfloat16), w2_ref[...],
                        preferred_element_type=jnp.float32)
            ridx = i * _BS + jax.lax.broadcasted_iota(jnp.int32, z.shape, 0)
            zb = jnp.where(ridx < n, z, 0.0).astype(jnp.bfloat16)
            xz_ref[rows, h_dim:] = zb
            accs = tuple(
                out_ref[rows_q[q], :] + jnp.dot(
                    tiles[q], zb, preferred_element_type=jnp.float32)
                for q in range(_NQ))

            @pl.when(i < nb - 1)
            def _():
                for q in range(_NQ):
                    out_ref[rows_q[q], :] = accs[q]

            @pl.when(i == nb - 1)
            def _():
                for q in range(_NQ):
                    o = accs[q] + b2_ref[...]
                    m = jnp.max(o, axis=1, keepdims=True)
                    e = o - m
                    out_ref[rows_q[q], :] = e - jnp.log(
                        jnp.sum(jnp.exp(e), axis=1, keepdims=True))

    @pl.when(jnp.logical_not(sweep1))
    def _():
        zs = xz_ref[pl.ds(j * _BS, _BS), h_dim:]
        accs = tuple(
            out_ref[rows_q[q], :] + jnp.dot(
                tiles[q], zs, preferred_element_type=jnp.float32)
            for q in range(_NQ))

        @pl.when(j < nb - 1)
        def _():
            for q in range(_NQ):
                out_ref[rows_q[q], :] = accs[q]

        @pl.when(j == nb - 1)
        def _():
            for q in range(_NQ):
                o = accs[q] + b2_ref[...]
                m = jnp.max(o, axis=1, keepdims=True)
                e = o - m
                out_ref[rows_q[q], :] = e - jnp.log(
                    jnp.sum(jnp.exp(e), axis=1, keepdims=True))


@jax.jit
def kernel(feature, adj, W1, b1, W2, b2):
    n, f_in = feature.shape
    h_dim = W1.shape[1]
    c_dim = W2.shape[1]
    nb = -(-n // _BS)
    npad = nb * _BS

    im_l, jm_l = [], []
    for i in range(nb):
        for j in range(nb):
            if j != i:
                im_l.append(i)
                jm_l.append(j)
        im_l.append(i)
        jm_l.append(i)
    for i in range(nb):
        for j in range(i + 1, nb):
            im_l.append(i)
            jm_l.append(j)
    im = jnp.asarray(np.asarray(im_l, dtype=np.int32))
    jm = jnp.asarray(np.asarray(jm_l, dtype=np.int32))
    steps = int(im.shape[0])

    feature_p = jnp.pad(feature.astype(jnp.bfloat16), ((0, npad - n), (0, 0)))
    b1_2d = b1.reshape(1, h_dim)
    b2_2d = b2.reshape(1, c_dim)

    def _mk_spec(q):
        return pl.BlockSpec(
            (_QB, _BS),
            lambda t, im_, jm_, q=q: (_NQ * im_[t] + q, jm_[t]))

    grid_spec = pltpu.PrefetchScalarGridSpec(
        num_scalar_prefetch=2,
        grid=(steps,),
        in_specs=[_mk_spec(q) for q in range(_NQ)] + [
            pl.BlockSpec((npad, f_in), lambda t, im_, jm_: (0, 0)),
            pl.BlockSpec((f_in, h_dim), lambda t, im_, jm_: (0, 0)),
            pl.BlockSpec((1, h_dim), lambda t, im_, jm_: (0, 0)),
            pl.BlockSpec((h_dim, c_dim), lambda t, im_, jm_: (0, 0)),
            pl.BlockSpec((1, c_dim), lambda t, im_, jm_: (0, 0)),
        ],
        out_specs=pl.BlockSpec((npad, c_dim), lambda t, im_, jm_: (0, 0)),
        scratch_shapes=[
            pltpu.VMEM((npad, h_dim), jnp.float32),
            pltpu.VMEM((npad, h_dim + c_dim), jnp.bfloat16),
        ],
    )

    out = pl.pallas_call(
        functools.partial(_tri_kernel, nb=nb, n=n, h_dim=h_dim),
        grid_spec=grid_spec,
        out_shape=jax.ShapeDtypeStruct((npad, c_dim), jnp.float32),
        compiler_params=pltpu.CompilerParams(
            vmem_limit_bytes=100 * 1024 * 1024),
    )(im, jm, adj, adj, adj, adj, feature_p, W1.astype(jnp.bfloat16),
      b1_2d, W2.astype(jnp.bfloat16), b2_2d)

    return out[:n]
